# Initial kernel scaffold; baseline (speedup 1.0000x reference)
import jax
import jax.numpy as jnp
from jax.experimental import pallas as pl


def kernel(edge_index, edge_vec, edge_len, r_max, fc1_w1, fc1_w2, fc2_w1, fc2_w2):
    raise NotImplementedError("write your pallas kernel here")


# speedup vs baseline: 1.2037x; 1.2037x over previous
import functools
import math

import jax
import jax.numpy as jnp
from jax import lax
from jax.experimental import pallas as pl
from jax.experimental.pallas import tpu as pltpu
from jax.experimental.pallas import tpu_sc as plsc

N = 10000
NPAD = 10240
E = 160000
INV_SQRT_NN = float(1.0 / math.sqrt(3.8))
INV_SQRT3 = float(1.0 / math.sqrt(3.0))

NC = 2
NS = 16
NW = NC * NS
CHUNK = 128
NCHUNK = E // CHUNK
NJ_REM = NCHUNK - (NCHUNK // NW) * NW
NJ_REM_S = NCHUNK - (NCHUNK // NS) * NS
ZROWS = 40

_SC_MESH = dict(core_axis_name="c", subcore_axis_name="s")


_BE = 4000


def _edge_body(rmax_ref, vec_ref, len_ref, sh_ref, emb_ref):
    v = vec_ref[...]
    x = v[:, 0:1]
    y = v[:, 1:2]
    z = v[:, 2:3]
    r = jnp.sqrt(x * x + y * y + z * z)
    d = jnp.maximum(r, 1e-9)
    x = x / d
    y = y / d
    z = z / d
    s3 = math.sqrt(3.0)
    s15 = math.sqrt(15.0)
    sh = jnp.concatenate([
        jnp.ones_like(x),
        s3 * x, s3 * y, s3 * z,
        s15 * x * y,
        s15 * y * z,
        (math.sqrt(5.0) / 2.0) * (3.0 * z * z - 1.0),
        s15 * x * z,
        (s15 / 2.0) * (x * x - y * y),
        math.sqrt(35.0 / 8.0) * y * (3.0 * x * x - y * y),
        math.sqrt(105.0) * x * y * z,
        math.sqrt(21.0 / 8.0) * y * (4.0 * z * z - x * x - y * y),
        (math.sqrt(7.0) / 2.0) * z * (2.0 * z * z - 3.0 * x * x - 3.0 * y * y),
        math.sqrt(21.0 / 8.0) * x * (4.0 * z * z - x * x - y * y),
        (math.sqrt(105.0) / 2.0) * z * (x * x - y * y),
        math.sqrt(35.0 / 8.0) * x * (x * x - 3.0 * y * y),
    ], axis=1)
    sh_ref[...] = sh

    el = len_ref[...]
    rmax = rmax_ref[0, 0]
    step = rmax * 0.5
    cols = []
    for j in range(3):
        diff = (el - j * step) / step
        cols.append(jnp.exp(-(diff * diff)) * (1.0 / 1.12))
    emb_ref[...] = jnp.concatenate(cols, axis=1)


def _edge_features(edge_vec, edge_len2, rmax2):
    grid = E // _BE
    return pl.pallas_call(
        _edge_body,
        grid=(grid,),
        in_specs=[
            pl.BlockSpec((1, 1), lambda i: (0, 0)),
            pl.BlockSpec((_BE, 3), lambda i: (i, 0)),
            pl.BlockSpec((_BE, 1), lambda i: (i, 0)),
        ],
        out_specs=[
            pl.BlockSpec((_BE, 16), lambda i: (i, 0)),
            pl.BlockSpec((_BE, 3), lambda i: (i, 0)),
        ],
        out_shape=[
            jax.ShapeDtypeStruct((E, 16), jnp.float32),
            jax.ShapeDtypeStruct((E, 3), jnp.float32),
        ],
    )(rmax2, edge_vec, edge_len2)


_BT = 2000


def _tp1_body(emb_ref, g1a_ref, g1b_ref, sh_ref, w1_ref, w2_ref, ef_ref):
    emb = emb_ref[...]
    h = (emb[:, 0:1] * w1_ref[0:1, :]
         + emb[:, 1:2] * w1_ref[1:2, :]
         + emb[:, 2:3] * w1_ref[2:3, :])
    h = jax.nn.relu(h * INV_SQRT3)
    w1e = jnp.dot(h.astype(jnp.bfloat16), w2_ref[...].astype(jnp.bfloat16),
                  preferred_element_type=jnp.float32) * (1.0 / 16.0)
    g = (g1a_ref[...] + g1b_ref[...]) * INV_SQRT_NN
    prod = g * sh_ref[...]
    d0 = prod[:, 0:1]
    d1 = jnp.sum(prod[:, 1:4], axis=1, keepdims=True)
    d2 = jnp.sum(prod[:, 4:9], axis=1, keepdims=True)
    d3 = jnp.sum(prod[:, 9:16], axis=1, keepdims=True)
    ef = (d0 * w1e[:, 0:96] + d1 * w1e[:, 96:192]
          + d2 * w1e[:, 192:288] + d3 * w1e[:, 288:384]) * 0.5
    ef_ref[...] = jnp.concatenate(
        [ef, jnp.zeros((ef.shape[0], 32), jnp.float32)], axis=1)


def _tp1_fused(emb, g1a, g1b, sh, fc1_w1, fc1_w2):
    grid = E // _BT
    return pl.pallas_call(
        _tp1_body,
        grid=(grid,),
        in_specs=[
            pl.BlockSpec((_BT, 3), lambda i: (i, 0)),
            pl.BlockSpec((_BT, 16), lambda i: (i, 0)),
            pl.BlockSpec((_BT, 16), lambda i: (i, 0)),
            pl.BlockSpec((_BT, 16), lambda i: (i, 0)),
            pl.BlockSpec((3, 256), lambda i: (0, 0)),
            pl.BlockSpec((256, 384), lambda i: (0, 0)),
        ],
        out_specs=pl.BlockSpec((_BT, 128), lambda i: (i, 0)),
        out_shape=jax.ShapeDtypeStruct((E, 128), jnp.float32),
    )(emb, g1a, g1b, sh, fc1_w1, fc1_w2)


_BN = 2000


def _gate_body(xa_ref, xb_ref, out_ref):
    x = (xa_ref[:, 0:96] + xb_ref[:, 0:96]) * INV_SQRT_NN
    scalars = jnp.concatenate(
        [jax.nn.relu(x[:, 0:16]), jnp.abs(x[:, 16:32])], axis=1)
    g = x[:, 32:64]
    gates = jnp.concatenate([
        jax.nn.relu(g[:, 0:8]), jnp.tanh(g[:, 8:16]),
        jax.nn.relu(g[:, 16:24]), jnp.tanh(g[:, 24:32])], axis=1)
    xg = jnp.concatenate([scalars, gates * x[:, 64:96]], axis=1)
    out_ref[...] = jnp.concatenate(
        [xg, jnp.zeros((xg.shape[0], 64), jnp.float32)], axis=1)


def _gate(x2a, x2b):
    grid = N // _BN
    return pl.pallas_call(
        _gate_body,
        grid=(grid,),
        in_specs=[
            pl.BlockSpec((_BN, 128), lambda i: (i, 0)),
            pl.BlockSpec((_BN, 128), lambda i: (i, 0)),
        ],
        out_specs=pl.BlockSpec((_BN, 128), lambda i: (i, 0)),
        out_shape=jax.ShapeDtypeStruct((N, 128), jnp.float32),
    )(x2a, x2b)


def _tp2_body(emb_ref, g2_ref, w1_ref, w2_ref, ef2_ref):
    emb = emb_ref[...]
    h = (emb[:, 0:1] * w1_ref[0:1, :]
         + emb[:, 1:2] * w1_ref[1:2, :]
         + emb[:, 2:3] * w1_ref[2:3, :])
    h = jax.nn.relu(h * INV_SQRT3)
    w2e = jnp.dot(h.astype(jnp.bfloat16), w2_ref[...].astype(jnp.bfloat16),
                  preferred_element_type=jnp.float32) * (1.0 / 16.0)
    s = jnp.sum(g2_ref[...] * w2e, axis=1, keepdims=True)
    ef2_ref[...] = s * (0.125 * INV_SQRT_NN)


def _tp2_fused(emb, g2, fc2_w1, fc2_w2):
    grid = E // _BT
    return pl.pallas_call(
        _tp2_body,
        grid=(grid,),
        in_specs=[
            pl.BlockSpec((_BT, 3), lambda i: (i, 0)),
            pl.BlockSpec((_BT, 64), lambda i: (i, 0)),
            pl.BlockSpec((3, 256), lambda i: (0, 0)),
            pl.BlockSpec((256, 64), lambda i: (0, 0)),
        ],
        out_specs=pl.BlockSpec((_BT, 1), lambda i: (i, 0)),
        out_shape=jax.ShapeDtypeStruct((E, 1), jnp.float32),
    )(emb, g2, fc2_w1, fc2_w2)


def _zero_table_128(table, zbuf, s):
    zero16 = jnp.zeros((16,), jnp.float32)

    @pl.when(s < 10)
    def _():
        def zfill(i, _):
            r = i // 8
            k = i % 8
            zbuf[r, pl.ds(k * 16, 16)] = zero16
            return 0

        lax.fori_loop(0, ZROWS * 8, zfill, 0)
        for k in range(1000 // ZROWS):
            pltpu.sync_copy(zbuf, table.at[pl.ds(s * 1000 + k * ZROWS, ZROWS), :])


def _stage_idx(idx_hbm, base, idx_flat, idx2):
    pltpu.sync_copy(idx_hbm.at[pl.ds(base, CHUNK)], idx_flat)
    for k in range(CHUNK // 16):
        idx2[0, pl.ds(k * 16, 16)] = idx_flat[pl.ds(k * 16, 16)]


@functools.lru_cache(maxsize=None)
def _make_scatter_gather16():
  @functools.partial(
      pl.kernel,
      out_type=(jax.ShapeDtypeStruct((E, 16), jnp.float32),
                jax.ShapeDtypeStruct((E, 16), jnp.float32)),
      mesh=plsc.VectorSubcoreMesh(**_SC_MESH),
      scratch_types=[
          pltpu.VMEM_SHARED((N, 128), jnp.float32),
          pltpu.VMEM((CHUNK,), jnp.int32),
          pltpu.VMEM((1, CHUNK), jnp.int32),
          pltpu.VMEM((16, 128), jnp.float32),
          pltpu.VMEM((CHUNK, 128), jnp.float32),
          pltpu.VMEM((CHUNK, 16), jnp.float32),
          pltpu.VMEM((ZROWS, 128), jnp.float32),
          pltpu.SemaphoreType.DMA,
      ],
  )
  def sg(dst_hbm, src_hbm, sh8_hbm, o0_hbm, o1_hbm,
         table, idx_flat, idx2, data_c, data_v, out_v, zbuf, sem):
    c = lax.axis_index("c")
    s = lax.axis_index("s")
    wid = s * NC + c
    zero16 = jnp.zeros((16,), jnp.float32)

    def dz(i, _):
        r = i // 7
        k = i % 7
        data_v[r, pl.ds(16 + k * 16, 16)] = zero16
        return 0

    lax.fori_loop(0, CHUNK * 7, dz, 0)

    _zero_table_128(table, zbuf, s)
    plsc.subcore_barrier()

    nj = jnp.where(wid < NJ_REM, NCHUNK // NW + 1, NCHUNK // NW)

    def body(j, _):
        base = (j * NW + wid) * CHUNK
        _stage_idx(dst_hbm, base, idx_flat, idx2)
        base8 = pl.multiple_of(base // 8, 16)
        pltpu.sync_copy(sh8_hbm.at[pl.ds(base8, CHUNK // 8), :], data_c)

        def expand(i, _):
            data_v[i, pl.ds(0, 16)] = data_c[i // 8, pl.ds((i % 8) * 16, 16)]
            return 0

        lax.fori_loop(0, CHUNK, expand, 0)
        pltpu.async_copy(data_v, table.at[idx2.at[0]], sem, add=True).wait()
        return 0

    lax.fori_loop(0, nj, body, 0)
    plsc.subcore_barrier()

    njg = jnp.where(s < NJ_REM_S, NCHUNK // NS + 1, NCHUNK // NS)

    def gbody(j, _):
        base = (j * NS + s) * CHUNK
        pltpu.sync_copy(src_hbm.at[pl.ds(base, CHUNK)], idx_flat)
        pltpu.async_copy(table.at[idx_flat], data_v, sem).wait()

        def compact(i, _):
            out_v[i, :] = data_v[i, pl.ds(0, 16)]
            return 0

        lax.fori_loop(0, CHUNK, compact, 0)

        @pl.when(c == 0)
        def _():
            pltpu.sync_copy(out_v, o0_hbm.at[pl.ds(base, CHUNK), :])

        @pl.when(c == 1)
        def _():
            pltpu.sync_copy(out_v, o1_hbm.at[pl.ds(base, CHUNK), :])

        return 0

    lax.fori_loop(0, njg, gbody, 0)

  return sg


def _scatter_gather16(dst, src, sh8):
    return _make_scatter_gather16()(dst, src, sh8)


@functools.lru_cache(maxsize=None)
def _make_scatter96():
  @functools.partial(
      pl.kernel,
      out_type=(jax.ShapeDtypeStruct((N, 128), jnp.float32),
                jax.ShapeDtypeStruct((N, 128), jnp.float32)),
      mesh=plsc.VectorSubcoreMesh(**_SC_MESH),
      scratch_types=[
          pltpu.VMEM_SHARED((N, 128), jnp.float32),
          pltpu.VMEM((CHUNK,), jnp.int32),
          pltpu.VMEM((1, CHUNK), jnp.int32),
          pltpu.VMEM((CHUNK, 128), jnp.float32),
          pltpu.VMEM((ZROWS, 128), jnp.float32),
          pltpu.SemaphoreType.DMA,
      ],
  )
  def scat(dst_hbm, ef_hbm, o0_hbm, o1_hbm,
           table, idx_flat, idx2, data_v, zbuf, sem):
    c = lax.axis_index("c")
    s = lax.axis_index("s")
    wid = s * NC + c

    _zero_table_128(table, zbuf, s)
    plsc.subcore_barrier()

    nj = jnp.where(wid < NJ_REM, NCHUNK // NW + 1, NCHUNK // NW)

    def body(j, _):
        base = (j * NW + wid) * CHUNK
        _stage_idx(dst_hbm, base, idx_flat, idx2)
        pltpu.sync_copy(ef_hbm.at[pl.ds(base, CHUNK), :], data_v)
        pltpu.async_copy(data_v, table.at[idx2.at[0]], sem, add=True).wait()
        return 0

    lax.fori_loop(0, nj, body, 0)
    plsc.subcore_barrier()

    @pl.when(jnp.logical_and(c == 0, s < 10))
    def _():
        pltpu.sync_copy(table.at[pl.ds(s * 1000, 1000), :],
                        o0_hbm.at[pl.ds(s * 1000, 1000), :])

    @pl.when(jnp.logical_and(c == 1, s < 10))
    def _():
        pltpu.sync_copy(table.at[pl.ds(s * 1000, 1000), :],
                        o1_hbm.at[pl.ds(s * 1000, 1000), :])

  return scat


def _scatter96(dst, ef):
    return _make_scatter96()(dst, ef)


@functools.lru_cache(maxsize=None)
def _make_gather64():
  @functools.partial(
      pl.kernel,
      out_type=jax.ShapeDtypeStruct((E, 64), jnp.float32),
      mesh=plsc.VectorSubcoreMesh(**_SC_MESH),
      scratch_types=[
          pltpu.VMEM_SHARED((N, 128), jnp.float32),
          pltpu.VMEM((CHUNK,), jnp.int32),
          pltpu.VMEM((CHUNK, 128), jnp.float32),
          pltpu.VMEM((CHUNK, 64), jnp.float32),
          pltpu.SemaphoreType.DMA,
      ],
  )
  def gat(src_hbm, xg_hbm, o_hbm, table, idx_flat, rows, out_v, sem):
    c = lax.axis_index("c")
    s = lax.axis_index("s")
    wid = s * NC + c

    @pl.when(s < 10)
    def _():
        pltpu.sync_copy(xg_hbm.at[pl.ds(s * 1000, 1000), :],
                        table.at[pl.ds(s * 1000, 1000), :])

    plsc.subcore_barrier()

    nj = jnp.where(wid < NJ_REM, NCHUNK // NW + 1, NCHUNK // NW)

    def body(j, _):
        base = (j * NW + wid) * CHUNK
        pltpu.sync_copy(src_hbm.at[pl.ds(base, CHUNK)], idx_flat)
        pltpu.async_copy(table.at[idx_flat], rows, sem).wait()

        def compact(i, _):
            for k in range(4):
                out_v[i, pl.ds(k * 16, 16)] = rows[i, pl.ds(k * 16, 16)]
            return 0

        lax.fori_loop(0, CHUNK, compact, 0)
        pltpu.sync_copy(out_v, o_hbm.at[pl.ds(base, CHUNK), :])
        return 0

    lax.fori_loop(0, nj, body, 0)

  return gat


def _gather64(src, xg):
    return _make_gather64()(src, xg)


@functools.lru_cache(maxsize=None)
def _make_scatter1():
  @functools.partial(
      pl.kernel,
      out_type=jax.ShapeDtypeStruct((NPAD,), jnp.float32),
      mesh=plsc.VectorSubcoreMesh(**_SC_MESH),
      scratch_types=[
          pltpu.VMEM_SHARED((NPAD,), jnp.float32),
          pltpu.VMEM((CHUNK,), jnp.int32),
          pltpu.VMEM((1, CHUNK), jnp.int32),
          pltpu.VMEM((CHUNK,), jnp.float32),
          pltpu.VMEM((NPAD // NS,), jnp.float32),
          pltpu.SemaphoreType.DMA,
      ],
  )
  def scat1(dst_hbm, data_hbm, out_hbm, table, idx_flat, idx2, data_v, zbuf, sem):
    c = lax.axis_index("c")
    s = lax.axis_index("s")
    zero16 = jnp.zeros((16,), jnp.float32)
    zn = NPAD // NS

    @pl.when(c == 0)
    def _():
        def zfill(i, _):
            zbuf[pl.ds(i * 16, 16)] = zero16
            return 0

        lax.fori_loop(0, zn // 16, zfill, 0)
        pltpu.sync_copy(zbuf, table.at[pl.ds(s * zn, zn)])

    plsc.subcore_barrier()

    @pl.when(c == 0)
    def _():
        nj = jnp.where(s < NJ_REM_S, NCHUNK // NS + 1, NCHUNK // NS)

        def body(j, _):
            base = (j * NS + s) * CHUNK
            _stage_idx(dst_hbm, base, idx_flat, idx2)
            pltpu.sync_copy(data_hbm.at[pl.ds(base, CHUNK)], data_v)
            pltpu.async_copy(data_v, table.at[idx2.at[0]], sem, add=True).wait()
            return 0

        lax.fori_loop(0, nj, body, 0)

    plsc.subcore_barrier()

    @pl.when(c == 0)
    def _():
        pltpu.sync_copy(table.at[pl.ds(s * zn, zn)], zbuf)
        pltpu.sync_copy(zbuf, out_hbm.at[pl.ds(s * zn, zn)])

  return scat1


def _scatter1(dst, vals):
    return _make_scatter1()(dst, vals)


def kernel(edge_index, edge_vec, edge_len, r_max, fc1_w1, fc1_w2,
           fc2_w1, fc2_w2):
    src = edge_index[0]
    dst = edge_index[1]
    sh, emb = _edge_features(edge_vec, edge_len.reshape(E, 1),
                             r_max.reshape(1, 1))
    sh8 = sh.reshape(E // 8, 128)
    g1a, g1b = _scatter_gather16(dst, src, sh8)
    ef = _tp1_fused(emb, g1a, g1b, sh, fc1_w1, fc1_w2)
    x2a, x2b = _scatter96(dst, ef)
    xg = _gate(x2a, x2b)
    g2 = _gather64(src, xg)
    ef2 = _tp2_fused(emb, g2, fc2_w1, fc2_w2)
    out = _scatter1(dst, ef2.reshape(E))
    return out[:N].reshape(N, 1)


# speedup vs baseline: 2.0963x; 1.7415x over previous
import functools
import math

import jax
import jax.numpy as jnp
from jax import lax
from jax.experimental import pallas as pl
from jax.experimental.pallas import tpu as pltpu
from jax.experimental.pallas import tpu_sc as plsc

N = 10000
NPAD = 10240
E = 160000
INV_SQRT_NN = float(1.0 / math.sqrt(3.8))
INV_SQRT3 = float(1.0 / math.sqrt(3.0))

NC = 2
NS = 16
NW = NC * NS
CHUNK = 128
NCHUNK = E // CHUNK
NJ_REM = NCHUNK - (NCHUNK // NW) * NW
NJ_REM_S = NCHUNK - (NCHUNK // NS) * NS
ZROWS = 40

_SC_MESH = dict(core_axis_name="c", subcore_axis_name="s")


_BE = 16000


def _edge_body(rmax_ref, vec_ref, len_ref, sh_ref, emb_ref):
    v = vec_ref[...]
    x = v[0:1, :]
    y = v[1:2, :]
    z = v[2:3, :]
    r = jnp.sqrt(x * x + y * y + z * z)
    d = jnp.maximum(r, 1e-9)
    x = x / d
    y = y / d
    z = z / d
    s3 = math.sqrt(3.0)
    s15 = math.sqrt(15.0)
    sh = jnp.concatenate([
        jnp.ones_like(x),
        s3 * x, s3 * y, s3 * z,
        s15 * x * y,
        s15 * y * z,
        (math.sqrt(5.0) / 2.0) * (3.0 * z * z - 1.0),
        s15 * x * z,
        (s15 / 2.0) * (x * x - y * y),
        math.sqrt(35.0 / 8.0) * y * (3.0 * x * x - y * y),
        math.sqrt(105.0) * x * y * z,
        math.sqrt(21.0 / 8.0) * y * (4.0 * z * z - x * x - y * y),
        (math.sqrt(7.0) / 2.0) * z * (2.0 * z * z - 3.0 * x * x - 3.0 * y * y),
        math.sqrt(21.0 / 8.0) * x * (4.0 * z * z - x * x - y * y),
        (math.sqrt(105.0) / 2.0) * z * (x * x - y * y),
        math.sqrt(35.0 / 8.0) * x * (x * x - 3.0 * y * y),
    ], axis=0)
    sh_ref[...] = sh

    el = len_ref[...]
    rmax = rmax_ref[0, 0]
    step = rmax * 0.5
    cols = []
    for j in range(3):
        diff = (el - j * step) / step
        cols.append(jnp.exp(-(diff * diff)) * (1.0 / 1.12))
    emb_ref[...] = jnp.concatenate(cols, axis=0)


def _edge_features(edge_vecT, edge_lenT, rmax2):
    grid = E // _BE
    return pl.pallas_call(
        _edge_body,
        grid=(grid,),
        in_specs=[
            pl.BlockSpec((1, 1), lambda i: (0, 0)),
            pl.BlockSpec((3, _BE), lambda i: (0, i)),
            pl.BlockSpec((1, _BE), lambda i: (0, i)),
        ],
        out_specs=[
            pl.BlockSpec((16, _BE), lambda i: (0, i)),
            pl.BlockSpec((3, _BE), lambda i: (0, i)),
        ],
        out_shape=[
            jax.ShapeDtypeStruct((16, E), jnp.float32),
            jax.ShapeDtypeStruct((3, E), jnp.float32),
        ],
    )(rmax2, edge_vecT, edge_lenT)


_BT = 2000


def _tp1_body(emb_ref, g1a_ref, g1b_ref, sh_ref, w1_ref, w2_ref, ef_ref):
    emb = emb_ref[...]
    h = (emb[:, 0:1] * w1_ref[0:1, :]
         + emb[:, 1:2] * w1_ref[1:2, :]
         + emb[:, 2:3] * w1_ref[2:3, :])
    h = jax.nn.relu(h * INV_SQRT3)
    w1e = jnp.dot(h.astype(jnp.bfloat16), w2_ref[...].astype(jnp.bfloat16),
                  preferred_element_type=jnp.float32) * (1.0 / 16.0)
    g = (g1a_ref[...] + g1b_ref[...]) * INV_SQRT_NN
    prod = g * sh_ref[...]
    d0 = prod[:, 0:1]
    d1 = jnp.sum(prod[:, 1:4], axis=1, keepdims=True)
    d2 = jnp.sum(prod[:, 4:9], axis=1, keepdims=True)
    d3 = jnp.sum(prod[:, 9:16], axis=1, keepdims=True)
    ef = (d0 * w1e[:, 0:96] + d1 * w1e[:, 96:192]
          + d2 * w1e[:, 192:288] + d3 * w1e[:, 288:384]) * 0.5
    ef_ref[...] = jnp.concatenate(
        [ef, jnp.zeros((ef.shape[0], 32), jnp.float32)], axis=1)


def _tp1_fused(emb, g1a, g1b, sh, fc1_w1, fc1_w2):
    grid = E // _BT
    return pl.pallas_call(
        _tp1_body,
        grid=(grid,),
        in_specs=[
            pl.BlockSpec((_BT, 3), lambda i: (i, 0)),
            pl.BlockSpec((_BT, 16), lambda i: (i, 0)),
            pl.BlockSpec((_BT, 16), lambda i: (i, 0)),
            pl.BlockSpec((_BT, 16), lambda i: (i, 0)),
            pl.BlockSpec((3, 256), lambda i: (0, 0)),
            pl.BlockSpec((256, 384), lambda i: (0, 0)),
        ],
        out_specs=pl.BlockSpec((_BT, 128), lambda i: (i, 0)),
        out_shape=jax.ShapeDtypeStruct((E, 128), jnp.float32),
    )(emb, g1a, g1b, sh, fc1_w1, fc1_w2)


_BN = 2000


def _gate_body(xa_ref, xb_ref, out_ref):
    x = (xa_ref[:, 0:96] + xb_ref[:, 0:96]) * INV_SQRT_NN
    scalars = jnp.concatenate(
        [jax.nn.relu(x[:, 0:16]), jnp.abs(x[:, 16:32])], axis=1)
    g = x[:, 32:64]
    gates = jnp.concatenate([
        jax.nn.relu(g[:, 0:8]), jnp.tanh(g[:, 8:16]),
        jax.nn.relu(g[:, 16:24]), jnp.tanh(g[:, 24:32])], axis=1)
    xg = jnp.concatenate([scalars, gates * x[:, 64:96]], axis=1)
    out_ref[...] = jnp.concatenate(
        [xg, jnp.zeros((xg.shape[0], 64), jnp.float32)], axis=1)


def _gate(x2a, x2b):
    grid = N // _BN
    return pl.pallas_call(
        _gate_body,
        grid=(grid,),
        in_specs=[
            pl.BlockSpec((_BN, 128), lambda i: (i, 0)),
            pl.BlockSpec((_BN, 128), lambda i: (i, 0)),
        ],
        out_specs=pl.BlockSpec((_BN, 128), lambda i: (i, 0)),
        out_shape=jax.ShapeDtypeStruct((N, 128), jnp.float32),
    )(x2a, x2b)


def _tp2_body(emb_ref, g2_ref, w1_ref, w2_ref, ef2_ref):
    emb = emb_ref[...]
    h = (emb[:, 0:1] * w1_ref[0:1, :]
         + emb[:, 1:2] * w1_ref[1:2, :]
         + emb[:, 2:3] * w1_ref[2:3, :])
    h = jax.nn.relu(h * INV_SQRT3)
    w2e = jnp.dot(h.astype(jnp.bfloat16), w2_ref[...].astype(jnp.bfloat16),
                  preferred_element_type=jnp.float32) * (1.0 / 16.0)
    s = jnp.sum(g2_ref[...] * w2e, axis=1, keepdims=True)
    ef2_ref[...] = s * (0.125 * INV_SQRT_NN)


def _tp2_fused(emb, g2, fc2_w1, fc2_w2):
    grid = E // _BT
    return pl.pallas_call(
        _tp2_body,
        grid=(grid,),
        in_specs=[
            pl.BlockSpec((_BT, 3), lambda i: (i, 0)),
            pl.BlockSpec((_BT, 64), lambda i: (i, 0)),
            pl.BlockSpec((3, 256), lambda i: (0, 0)),
            pl.BlockSpec((256, 64), lambda i: (0, 0)),
        ],
        out_specs=pl.BlockSpec((_BT, 1), lambda i: (i, 0)),
        out_shape=jax.ShapeDtypeStruct((E, 1), jnp.float32),
    )(emb, g2, fc2_w1, fc2_w2)


def _zero_table_128(table, zbuf, s):
    zero16 = jnp.zeros((16,), jnp.float32)

    @pl.when(s < 10)
    def _():
        def zfill(r, _):
            for k in range(8):
                zbuf[r, pl.ds(k * 16, 16)] = zero16
            return 0

        lax.fori_loop(0, ZROWS, zfill, 0)
        for k in range(1000 // ZROWS):
            pltpu.sync_copy(zbuf, table.at[pl.ds(s * 1000 + k * ZROWS, ZROWS), :])


def _stage_idx(idx_hbm, base, idx_flat, idx2):
    pltpu.sync_copy(idx_hbm.at[pl.ds(base, CHUNK)], idx_flat)
    for k in range(CHUNK // 16):
        idx2[0, pl.ds(k * 16, 16)] = idx_flat[pl.ds(k * 16, 16)]


@functools.lru_cache(maxsize=None)
def _make_scatter_gather16():
  @functools.partial(
      pl.kernel,
      out_type=(jax.ShapeDtypeStruct((E, 16), jnp.float32),
                jax.ShapeDtypeStruct((E, 16), jnp.float32)),
      mesh=plsc.VectorSubcoreMesh(**_SC_MESH),
      scratch_types=[
          pltpu.VMEM_SHARED((N, 128), jnp.float32),
          pltpu.VMEM((CHUNK,), jnp.int32),
          pltpu.VMEM((1, CHUNK), jnp.int32),
          pltpu.VMEM((16, 128), jnp.float32),
          pltpu.VMEM((CHUNK, 128), jnp.float32),
          pltpu.VMEM((CHUNK, 16), jnp.float32),
          pltpu.VMEM((ZROWS, 128), jnp.float32),
          pltpu.SemaphoreType.DMA,
      ],
  )
  def sg(dst_hbm, src_hbm, sh8_hbm, o0_hbm, o1_hbm,
         table, idx_flat, idx2, data_c, data_v, out_v, zbuf, sem):
    c = lax.axis_index("c")
    s = lax.axis_index("s")
    wid = s * NC + c
    zero16 = jnp.zeros((16,), jnp.float32)

    def dz(r, _):
        for k in range(7):
            data_v[r, pl.ds(16 + k * 16, 16)] = zero16
        return 0

    lax.fori_loop(0, CHUNK, dz, 0)

    _zero_table_128(table, zbuf, s)
    plsc.subcore_barrier()

    nj = jnp.where(wid < NJ_REM, NCHUNK // NW + 1, NCHUNK // NW)

    def body(j, _):
        base = (j * NW + wid) * CHUNK
        _stage_idx(dst_hbm, base, idx_flat, idx2)
        base8 = pl.multiple_of(base // 8, 16)
        pltpu.sync_copy(sh8_hbm.at[pl.ds(base8, CHUNK // 8), :], data_c)
        for r in range(CHUNK // 8):
            for k in range(8):
                data_v[r * 8 + k, pl.ds(0, 16)] = data_c[r, pl.ds(k * 16, 16)]
        pltpu.async_copy(data_v, table.at[idx2.at[0]], sem, add=True).wait()
        return 0

    lax.fori_loop(0, nj, body, 0)
    plsc.subcore_barrier()

    njg = jnp.where(s < NJ_REM_S, NCHUNK // NS + 1, NCHUNK // NS)

    def gbody(j, _):
        base = (j * NS + s) * CHUNK
        pltpu.sync_copy(src_hbm.at[pl.ds(base, CHUNK)], idx_flat)
        pltpu.async_copy(table.at[idx_flat], data_v, sem).wait()
        for i in range(CHUNK):
            out_v[i, :] = data_v[i, pl.ds(0, 16)]

        @pl.when(c == 0)
        def _():
            pltpu.sync_copy(out_v, o0_hbm.at[pl.ds(base, CHUNK), :])

        @pl.when(c == 1)
        def _():
            pltpu.sync_copy(out_v, o1_hbm.at[pl.ds(base, CHUNK), :])

        return 0

    lax.fori_loop(0, njg, gbody, 0)

  return sg


def _scatter_gather16(dst, src, sh8):
    return _make_scatter_gather16()(dst, src, sh8)


@functools.lru_cache(maxsize=None)
def _make_scatter96():
  @functools.partial(
      pl.kernel,
      out_type=(jax.ShapeDtypeStruct((N, 128), jnp.float32),
                jax.ShapeDtypeStruct((N, 128), jnp.float32)),
      mesh=plsc.VectorSubcoreMesh(**_SC_MESH),
      scratch_types=[
          pltpu.VMEM_SHARED((N, 128), jnp.float32),
          pltpu.VMEM((CHUNK,), jnp.int32),
          pltpu.VMEM((1, CHUNK), jnp.int32),
          pltpu.VMEM((CHUNK, 128), jnp.float32),
          pltpu.VMEM((ZROWS, 128), jnp.float32),
          pltpu.SemaphoreType.DMA,
      ],
  )
  def scat(dst_hbm, ef_hbm, o0_hbm, o1_hbm,
           table, idx_flat, idx2, data_v, zbuf, sem):
    c = lax.axis_index("c")
    s = lax.axis_index("s")
    wid = s * NC + c

    _zero_table_128(table, zbuf, s)
    plsc.subcore_barrier()

    nj = jnp.where(wid < NJ_REM, NCHUNK // NW + 1, NCHUNK // NW)

    def body(j, _):
        base = (j * NW + wid) * CHUNK
        _stage_idx(dst_hbm, base, idx_flat, idx2)
        pltpu.sync_copy(ef_hbm.at[pl.ds(base, CHUNK), :], data_v)
        pltpu.async_copy(data_v, table.at[idx2.at[0]], sem, add=True).wait()
        return 0

    lax.fori_loop(0, nj, body, 0)
    plsc.subcore_barrier()

    @pl.when(jnp.logical_and(c == 0, s < 10))
    def _():
        pltpu.sync_copy(table.at[pl.ds(s * 1000, 1000), :],
                        o0_hbm.at[pl.ds(s * 1000, 1000), :])

    @pl.when(jnp.logical_and(c == 1, s < 10))
    def _():
        pltpu.sync_copy(table.at[pl.ds(s * 1000, 1000), :],
                        o1_hbm.at[pl.ds(s * 1000, 1000), :])

  return scat


def _scatter96(dst, ef):
    return _make_scatter96()(dst, ef)


@functools.lru_cache(maxsize=None)
def _make_gather64():
  @functools.partial(
      pl.kernel,
      out_type=jax.ShapeDtypeStruct((E, 64), jnp.float32),
      mesh=plsc.VectorSubcoreMesh(**_SC_MESH),
      scratch_types=[
          pltpu.VMEM_SHARED((N, 128), jnp.float32),
          pltpu.VMEM((CHUNK,), jnp.int32),
          pltpu.VMEM((CHUNK, 128), jnp.float32),
          pltpu.VMEM((CHUNK, 64), jnp.float32),
          pltpu.SemaphoreType.DMA,
      ],
  )
  def gat(src_hbm, xg_hbm, o_hbm, table, idx_flat, rows, out_v, sem):
    c = lax.axis_index("c")
    s = lax.axis_index("s")
    wid = s * NC + c

    @pl.when(s < 10)
    def _():
        pltpu.sync_copy(xg_hbm.at[pl.ds(s * 1000, 1000), :],
                        table.at[pl.ds(s * 1000, 1000), :])

    plsc.subcore_barrier()

    nj = jnp.where(wid < NJ_REM, NCHUNK // NW + 1, NCHUNK // NW)

    def body(j, _):
        base = (j * NW + wid) * CHUNK
        pltpu.sync_copy(src_hbm.at[pl.ds(base, CHUNK)], idx_flat)
        pltpu.async_copy(table.at[idx_flat], rows, sem).wait()

        def compact(i, _):
            for k in range(4):
                out_v[i, pl.ds(k * 16, 16)] = rows[i, pl.ds(k * 16, 16)]
            return 0

        lax.fori_loop(0, CHUNK, compact, 0)
        pltpu.sync_copy(out_v, o_hbm.at[pl.ds(base, CHUNK), :])
        return 0

    lax.fori_loop(0, nj, body, 0)

  return gat


def _gather64(src, xg):
    return _make_gather64()(src, xg)


@functools.lru_cache(maxsize=None)
def _make_scatter1():
  @functools.partial(
      pl.kernel,
      out_type=jax.ShapeDtypeStruct((NPAD,), jnp.float32),
      mesh=plsc.VectorSubcoreMesh(**_SC_MESH),
      scratch_types=[
          pltpu.VMEM_SHARED((NPAD,), jnp.float32),
          pltpu.VMEM((CHUNK,), jnp.int32),
          pltpu.VMEM((1, CHUNK), jnp.int32),
          pltpu.VMEM((CHUNK,), jnp.float32),
          pltpu.VMEM((NPAD // NS,), jnp.float32),
          pltpu.SemaphoreType.DMA,
      ],
  )
  def scat1(dst_hbm, data_hbm, out_hbm, table, idx_flat, idx2, data_v, zbuf, sem):
    c = lax.axis_index("c")
    s = lax.axis_index("s")
    zero16 = jnp.zeros((16,), jnp.float32)
    zn = NPAD // NS

    @pl.when(c == 0)
    def _():
        def zfill(i, _):
            zbuf[pl.ds(i * 16, 16)] = zero16
            return 0

        lax.fori_loop(0, zn // 16, zfill, 0)
        pltpu.sync_copy(zbuf, table.at[pl.ds(s * zn, zn)])

    plsc.subcore_barrier()

    @pl.when(c == 0)
    def _():
        nj = jnp.where(s < NJ_REM_S, NCHUNK // NS + 1, NCHUNK // NS)

        def body(j, _):
            base = (j * NS + s) * CHUNK
            _stage_idx(dst_hbm, base, idx_flat, idx2)
            pltpu.sync_copy(data_hbm.at[pl.ds(base, CHUNK)], data_v)
            pltpu.async_copy(data_v, table.at[idx2.at[0]], sem, add=True).wait()
            return 0

        lax.fori_loop(0, nj, body, 0)

    plsc.subcore_barrier()

    @pl.when(c == 0)
    def _():
        pltpu.sync_copy(table.at[pl.ds(s * zn, zn)], zbuf)
        pltpu.sync_copy(zbuf, out_hbm.at[pl.ds(s * zn, zn)])

  return scat1


def _scatter1(dst, vals):
    return _make_scatter1()(dst, vals)


def kernel(edge_index, edge_vec, edge_len, r_max, fc1_w1, fc1_w2,
           fc2_w1, fc2_w2):
    src = edge_index[0]
    dst = edge_index[1]
    shT, embT = _edge_features(edge_vec.T, edge_len.reshape(1, E),
                               r_max.reshape(1, 1))
    sh = shT.T
    emb = embT.T
    sh8 = sh.reshape(E // 8, 128)
    g1a, g1b = _scatter_gather16(dst, src, sh8)
    ef = _tp1_fused(emb, g1a, g1b, sh, fc1_w1, fc1_w2)
    x2a, x2b = _scatter96(dst, ef)
    xg = _gate(x2a, x2b)
    g2 = _gather64(src, xg)
    ef2 = _tp2_fused(emb, g2, fc2_w1, fc2_w2)
    out = _scatter1(dst, ef2.reshape(E))
    return out[:N].reshape(N, 1)


# speedup vs baseline: 2.5154x; 1.1999x over previous
import functools
import math

import jax
import jax.numpy as jnp
from jax import lax
from jax.experimental import pallas as pl
from jax.experimental.pallas import tpu as pltpu
from jax.experimental.pallas import tpu_sc as plsc

N = 10000
NPAD = 10240
E = 160000
INV_SQRT_NN = float(1.0 / math.sqrt(3.8))
INV_SQRT3 = float(1.0 / math.sqrt(3.0))

NC = 2
NS = 16
NW = NC * NS
CHUNK = 128
NCHUNK = E // CHUNK
NJ_W = NCHUNK // NW
NJ_REM = NCHUNK - NJ_W * NW
NJ_S = NCHUNK // NS
NJ_REM_S = NCHUNK - NJ_S * NS
ZROWS = 40

_SC_MESH = dict(core_axis_name="c", subcore_axis_name="s")


_BE = 16000


def _edge_body(rmax_ref, vec_ref, len_ref, sh_ref, emb_ref):
    v = vec_ref[...]
    x = v[0:1, :]
    y = v[1:2, :]
    z = v[2:3, :]
    r = jnp.sqrt(x * x + y * y + z * z)
    d = jnp.maximum(r, 1e-9)
    x = x / d
    y = y / d
    z = z / d
    s3 = math.sqrt(3.0)
    s15 = math.sqrt(15.0)
    sh = jnp.concatenate([
        jnp.ones_like(x),
        s3 * x, s3 * y, s3 * z,
        s15 * x * y,
        s15 * y * z,
        (math.sqrt(5.0) / 2.0) * (3.0 * z * z - 1.0),
        s15 * x * z,
        (s15 / 2.0) * (x * x - y * y),
        math.sqrt(35.0 / 8.0) * y * (3.0 * x * x - y * y),
        math.sqrt(105.0) * x * y * z,
        math.sqrt(21.0 / 8.0) * y * (4.0 * z * z - x * x - y * y),
        (math.sqrt(7.0) / 2.0) * z * (2.0 * z * z - 3.0 * x * x - 3.0 * y * y),
        math.sqrt(21.0 / 8.0) * x * (4.0 * z * z - x * x - y * y),
        (math.sqrt(105.0) / 2.0) * z * (x * x - y * y),
        math.sqrt(35.0 / 8.0) * x * (x * x - 3.0 * y * y),
    ], axis=0)
    sh_ref[...] = sh

    el = len_ref[...]
    rmax = rmax_ref[0, 0]
    step = rmax * 0.5
    cols = []
    for j in range(3):
        diff = (el - j * step) / step
        cols.append(jnp.exp(-(diff * diff)) * (1.0 / 1.12))
    emb_ref[...] = jnp.concatenate(cols, axis=0)


def _edge_features(edge_vecT, edge_lenT, rmax2):
    grid = E // _BE
    return pl.pallas_call(
        _edge_body,
        grid=(grid,),
        in_specs=[
            pl.BlockSpec((1, 1), lambda i: (0, 0)),
            pl.BlockSpec((3, _BE), lambda i: (0, i)),
            pl.BlockSpec((1, _BE), lambda i: (0, i)),
        ],
        out_specs=[
            pl.BlockSpec((16, _BE), lambda i: (0, i)),
            pl.BlockSpec((3, _BE), lambda i: (0, i)),
        ],
        out_shape=[
            jax.ShapeDtypeStruct((16, E), jnp.float32),
            jax.ShapeDtypeStruct((3, E), jnp.float32),
        ],
    )(rmax2, edge_vecT, edge_lenT)


_BT = 2000


def _tp1_body(emb_ref, g1a_ref, g1b_ref, sh_ref, w1_ref, w2_ref, ef_ref):
    emb = emb_ref[...]
    h = (emb[:, 0:1] * w1_ref[0:1, :]
         + emb[:, 1:2] * w1_ref[1:2, :]
         + emb[:, 2:3] * w1_ref[2:3, :])
    h = jax.nn.relu(h * INV_SQRT3)
    w1e = jnp.dot(h.astype(jnp.bfloat16), w2_ref[...].astype(jnp.bfloat16),
                  preferred_element_type=jnp.float32) * (1.0 / 16.0)
    g = (g1a_ref[...] + g1b_ref[...]) * INV_SQRT_NN
    prod = g * sh_ref[...]
    d0 = prod[:, 0:1]
    d1 = jnp.sum(prod[:, 1:4], axis=1, keepdims=True)
    d2 = jnp.sum(prod[:, 4:9], axis=1, keepdims=True)
    d3 = jnp.sum(prod[:, 9:16], axis=1, keepdims=True)
    ef = (d0 * w1e[:, 0:96] + d1 * w1e[:, 96:192]
          + d2 * w1e[:, 192:288] + d3 * w1e[:, 288:384]) * 0.5
    ef_ref[...] = jnp.concatenate(
        [ef, jnp.zeros((ef.shape[0], 32), jnp.float32)], axis=1)


def _tp1_fused(emb, g1a, g1b, sh, fc1_w1, fc1_w2):
    grid = E // _BT
    return pl.pallas_call(
        _tp1_body,
        grid=(grid,),
        in_specs=[
            pl.BlockSpec((_BT, 3), lambda i: (i, 0)),
            pl.BlockSpec((_BT, 16), lambda i: (i, 0)),
            pl.BlockSpec((_BT, 16), lambda i: (i, 0)),
            pl.BlockSpec((_BT, 16), lambda i: (i, 0)),
            pl.BlockSpec((3, 256), lambda i: (0, 0)),
            pl.BlockSpec((256, 384), lambda i: (0, 0)),
        ],
        out_specs=pl.BlockSpec((_BT, 128), lambda i: (i, 0)),
        out_shape=jax.ShapeDtypeStruct((E, 128), jnp.float32),
    )(emb, g1a, g1b, sh, fc1_w1, fc1_w2)


_BN = 2000


def _gate_body(xa_ref, xb_ref, out_ref):
    x = (xa_ref[:, 0:96] + xb_ref[:, 0:96]) * INV_SQRT_NN
    scalars = jnp.concatenate(
        [jax.nn.relu(x[:, 0:16]), jnp.abs(x[:, 16:32])], axis=1)
    g = x[:, 32:64]
    gates = jnp.concatenate([
        jax.nn.relu(g[:, 0:8]), jnp.tanh(g[:, 8:16]),
        jax.nn.relu(g[:, 16:24]), jnp.tanh(g[:, 24:32])], axis=1)
    xg = jnp.concatenate([scalars, gates * x[:, 64:96]], axis=1)
    out_ref[...] = jnp.concatenate(
        [xg, jnp.zeros((xg.shape[0], 64), jnp.float32)], axis=1)


def _gate(x2a, x2b):
    grid = N // _BN
    return pl.pallas_call(
        _gate_body,
        grid=(grid,),
        in_specs=[
            pl.BlockSpec((_BN, 128), lambda i: (i, 0)),
            pl.BlockSpec((_BN, 128), lambda i: (i, 0)),
        ],
        out_specs=pl.BlockSpec((_BN, 128), lambda i: (i, 0)),
        out_shape=jax.ShapeDtypeStruct((N, 128), jnp.float32),
    )(x2a, x2b)


def _tp2_body(emb_ref, g2_ref, w1_ref, w2_ref, ef2_ref):
    emb = emb_ref[...]
    h = (emb[:, 0:1] * w1_ref[0:1, :]
         + emb[:, 1:2] * w1_ref[1:2, :]
         + emb[:, 2:3] * w1_ref[2:3, :])
    h = jax.nn.relu(h * INV_SQRT3)
    w2e = jnp.dot(h.astype(jnp.bfloat16), w2_ref[...].astype(jnp.bfloat16),
                  preferred_element_type=jnp.float32) * (1.0 / 16.0)
    s = jnp.sum(g2_ref[...] * w2e, axis=1, keepdims=True)
    ef2_ref[...] = s * (0.125 * INV_SQRT_NN)


def _tp2_fused(emb, g2, fc2_w1, fc2_w2):
    grid = E // _BT
    return pl.pallas_call(
        _tp2_body,
        grid=(grid,),
        in_specs=[
            pl.BlockSpec((_BT, 3), lambda i: (i, 0)),
            pl.BlockSpec((_BT, 64), lambda i: (i, 0)),
            pl.BlockSpec((3, 256), lambda i: (0, 0)),
            pl.BlockSpec((256, 64), lambda i: (0, 0)),
        ],
        out_specs=pl.BlockSpec((_BT, 1), lambda i: (i, 0)),
        out_shape=jax.ShapeDtypeStruct((E, 1), jnp.float32),
    )(emb, g2, fc2_w1, fc2_w2)


def _zero_vmem_128(buf):
    zero16 = jnp.zeros((16,), jnp.float32)

    def zfill(r, _):
        for k in range(8):
            buf[r, pl.ds(k * 16, 16)] = zero16
        return 0

    lax.fori_loop(0, CHUNK, zfill, 0)


def _zero_table_from(table, zsrc, s):
    @pl.when(s < 10)
    def _():
        for k in range(7):
            pltpu.sync_copy(zsrc, table.at[pl.ds(s * 1000 + k * 128, 128), :])
        pltpu.sync_copy(zsrc.at[pl.ds(0, 104), :],
                        table.at[pl.ds(s * 1000 + 896, 104), :])


def _stage_idx_span(idx_hbm, idx_big, base0, n_full, extra):
    pltpu.sync_copy(idx_hbm.at[pl.ds(base0, n_full * CHUNK)],
                    idx_big.at[pl.ds(0, n_full * CHUNK)])

    @pl.when(extra)
    def _():
        pltpu.sync_copy(idx_hbm.at[pl.ds(base0 + n_full * CHUNK, CHUNK)],
                        idx_big.at[pl.ds(n_full * CHUNK, CHUNK)])


def _fill_idx2(idx_big, idx2, nj):
    def fill(j, _):
        for k in range(CHUNK // 16):
            idx2[j, pl.ds(k * 16, 16)] = idx_big[pl.ds(j * CHUNK + k * 16, 16)]
        return 0

    lax.fori_loop(0, nj, fill, 0)


@functools.lru_cache(maxsize=None)
def _make_scatter_gather16():
  @functools.partial(
      pl.kernel,
      out_type=(jax.ShapeDtypeStruct((E, 16), jnp.float32),
                jax.ShapeDtypeStruct((E, 16), jnp.float32)),
      mesh=plsc.VectorSubcoreMesh(**_SC_MESH),
      scratch_types=[
          pltpu.VMEM_SHARED((N, 128), jnp.float32),
          pltpu.VMEM((CHUNK,), jnp.int32),
          pltpu.VMEM((CHUNK,), jnp.int32),
          pltpu.VMEM((1, CHUNK), jnp.int32),
          pltpu.VMEM((16, 128), jnp.float32),
          pltpu.VMEM((CHUNK, 128), jnp.float32),
          pltpu.VMEM((CHUNK, 16), jnp.float32),
          pltpu.SemaphoreType.DMA,
          pltpu.SemaphoreType.DMA,
          pltpu.SemaphoreType.DMA,
          pltpu.SemaphoreType.DMA,
      ],
  )
  def sg(dst_hbm, src_hbm, sh8_hbm, o0_hbm, o1_hbm,
         table, ia, ib, idx2, dc, data_v, ov,
         sem_ia, sem_ib, sem_s, sem_w):
    c = lax.axis_index("c")
    s = lax.axis_index("s")
    wid = s * NC + c

    _zero_vmem_128(data_v)
    _zero_table_from(table, data_v, s)
    plsc.subcore_barrier()

    nj = jnp.where(wid < NJ_REM, NJ_W + 1, NJ_W)

    def start_idx(base, ibuf, sem_i):
        pltpu.async_copy(dst_hbm.at[pl.ds(base, CHUNK)], ibuf, sem_i)

    def do_scatter(j, ibuf, sem_i):
        base = pl.multiple_of((j * NW + wid) * CHUNK, CHUNK)
        pltpu.make_async_copy(dst_hbm.at[pl.ds(0, CHUNK)], ibuf, sem_i).wait()
        for k in range(CHUNK // 16):
            idx2[0, pl.ds(k * 16, 16)] = ibuf[pl.ds(k * 16, 16)]
        b8 = pl.multiple_of(base // 8, 16)
        pltpu.sync_copy(sh8_hbm.at[pl.ds(b8, CHUNK // 8), :], dc)
        for r in range(CHUNK // 8):
            for k in range(8):
                data_v[r * 8 + k, pl.ds(0, 16)] = dc[r, pl.ds(k * 16, 16)]
        pltpu.async_copy(data_v, table.at[idx2.at[0]], sem_s, add=True).wait()

    start_idx(wid * CHUNK, ia, sem_ia)

    def pbody(jj, _):
        j0 = jj * 2
        j1 = j0 + 1

        @pl.when(j1 < nj)
        def _():
            start_idx((j1 * NW + wid) * CHUNK, ib, sem_ib)

        do_scatter(j0, ia, sem_ia)

        @pl.when(j1 < nj)
        def _():
            @pl.when(j1 + 1 < nj)
            def _():
                start_idx(((j1 + 1) * NW + wid) * CHUNK, ia, sem_ia)

            do_scatter(j1, ib, sem_ib)

        return 0

    lax.fori_loop(0, (nj + 1) // 2, pbody, 0)
    plsc.subcore_barrier()

    njg = jnp.where(s < NJ_REM_S, NJ_S + 1, NJ_S)

    def gstart_idx(j, ibuf, sem_i):
        pltpu.async_copy(
            src_hbm.at[pl.ds((j * NS + s) * CHUNK, CHUNK)], ibuf, sem_i)

    def do_gather(j, ibuf, sem_i, first):
        base = pl.multiple_of((j * NS + s) * CHUNK, CHUNK)
        pltpu.make_async_copy(src_hbm.at[pl.ds(0, CHUNK)], ibuf, sem_i).wait()
        pltpu.async_copy(table.at[ibuf], data_v, sem_s).wait()

        @pl.when(jnp.logical_not(first))
        def _():
            pltpu.make_async_copy(ov, o0_hbm.at[pl.ds(0, CHUNK), :],
                                  sem_w).wait()

        for i in range(CHUNK):
            ov[i, :] = data_v[i, pl.ds(0, 16)]

        @pl.when(c == 0)
        def _():
            pltpu.async_copy(ov, o0_hbm.at[pl.ds(base, CHUNK), :], sem_w)

        @pl.when(c == 1)
        def _():
            pltpu.async_copy(ov, o1_hbm.at[pl.ds(base, CHUNK), :], sem_w)

    gstart_idx(0, ia, sem_ia)

    def gp(jj, _):
        j0 = jj * 2
        j1 = j0 + 1

        @pl.when(j1 < njg)
        def _():
            gstart_idx(j1, ib, sem_ib)

        do_gather(j0, ia, sem_ia, j0 == 0)

        @pl.when(j1 < njg)
        def _():
            @pl.when(j1 + 1 < njg)
            def _():
                gstart_idx(j1 + 1, ia, sem_ia)

            do_gather(j1, ib, sem_ib, jnp.bool_(False))

        return 0

    lax.fori_loop(0, (njg + 1) // 2, gp, 0)
    pltpu.make_async_copy(ov, o0_hbm.at[pl.ds(0, CHUNK), :], sem_w).wait()

  return sg


def _scatter_gather16(dst, src, sh8):
    return _make_scatter_gather16()(dst, src, sh8)


@functools.lru_cache(maxsize=None)
def _make_scatter96():
  @functools.partial(
      pl.kernel,
      out_type=(jax.ShapeDtypeStruct((N, 128), jnp.float32),
                jax.ShapeDtypeStruct((N, 128), jnp.float32)),
      mesh=plsc.VectorSubcoreMesh(**_SC_MESH),
      scratch_types=[
          pltpu.VMEM_SHARED((N, 128), jnp.float32),
          pltpu.VMEM((CHUNK,), jnp.int32),
          pltpu.VMEM((CHUNK,), jnp.int32),
          pltpu.VMEM((1, CHUNK), jnp.int32),
          pltpu.VMEM((CHUNK, 128), jnp.float32),
          pltpu.SemaphoreType.DMA,
          pltpu.SemaphoreType.DMA,
          pltpu.SemaphoreType.DMA,
      ],
  )
  def scat(dst_hbm, ef_hbm, o0_hbm, o1_hbm,
           table, ia, ib, idx2, data_v, sem_ia, sem_ib, sem_s):
    c = lax.axis_index("c")
    s = lax.axis_index("s")
    wid = s * NC + c

    _zero_vmem_128(data_v)
    _zero_table_from(table, data_v, s)
    plsc.subcore_barrier()

    nj = jnp.where(wid < NJ_REM, NJ_W + 1, NJ_W)

    def start_idx(j, ibuf, sem_i):
        pltpu.async_copy(
            dst_hbm.at[pl.ds((j * NW + wid) * CHUNK, CHUNK)], ibuf, sem_i)

    def do_scatter(j, ibuf, sem_i):
        base = pl.multiple_of((j * NW + wid) * CHUNK, CHUNK)
        pltpu.make_async_copy(dst_hbm.at[pl.ds(0, CHUNK)], ibuf, sem_i).wait()
        for k in range(CHUNK // 16):
            idx2[0, pl.ds(k * 16, 16)] = ibuf[pl.ds(k * 16, 16)]
        pltpu.sync_copy(ef_hbm.at[pl.ds(base, CHUNK), :], data_v)
        pltpu.async_copy(data_v, table.at[idx2.at[0]], sem_s, add=True).wait()

    start_idx(0, ia, sem_ia)

    def pbody(jj, _):
        j0 = jj * 2
        j1 = j0 + 1

        @pl.when(j1 < nj)
        def _():
            start_idx(j1, ib, sem_ib)

        do_scatter(j0, ia, sem_ia)

        @pl.when(j1 < nj)
        def _():
            @pl.when(j1 + 1 < nj)
            def _():
                start_idx(j1 + 1, ia, sem_ia)

            do_scatter(j1, ib, sem_ib)

        return 0

    lax.fori_loop(0, (nj + 1) // 2, pbody, 0)
    plsc.subcore_barrier()

    @pl.when(jnp.logical_and(c == 0, s < 10))
    def _():
        pltpu.sync_copy(table.at[pl.ds(s * 1000, 1000), :],
                        o0_hbm.at[pl.ds(s * 1000, 1000), :])

    @pl.when(jnp.logical_and(c == 1, s < 10))
    def _():
        pltpu.sync_copy(table.at[pl.ds(s * 1000, 1000), :],
                        o1_hbm.at[pl.ds(s * 1000, 1000), :])

  return scat


def _scatter96(dst, ef):
    return _make_scatter96()(dst, ef)


@functools.lru_cache(maxsize=None)
def _make_gather64():
  @functools.partial(
      pl.kernel,
      out_type=jax.ShapeDtypeStruct((E, 64), jnp.float32),
      mesh=plsc.VectorSubcoreMesh(**_SC_MESH),
      scratch_types=[
          pltpu.VMEM_SHARED((N, 128), jnp.float32),
          pltpu.VMEM((CHUNK,), jnp.int32),
          pltpu.VMEM((CHUNK,), jnp.int32),
          pltpu.VMEM((CHUNK, 128), jnp.float32),
          pltpu.VMEM((CHUNK, 64), jnp.float32),
          pltpu.SemaphoreType.DMA,
          pltpu.SemaphoreType.DMA,
          pltpu.SemaphoreType.DMA,
          pltpu.SemaphoreType.DMA,
      ],
  )
  def gat(src_hbm, xg_hbm, o_hbm, table, ia, ib, rows, ov,
          sem_ia, sem_ib, sem_g, sem_w):
    c = lax.axis_index("c")
    s = lax.axis_index("s")
    wid = s * NC + c

    @pl.when(s < 10)
    def _():
        pltpu.sync_copy(xg_hbm.at[pl.ds(s * 1000, 1000), :],
                        table.at[pl.ds(s * 1000, 1000), :])

    plsc.subcore_barrier()

    nj = jnp.where(wid < NJ_REM, NJ_W + 1, NJ_W)

    def start_idx(j, ibuf, sem_i):
        pltpu.async_copy(
            src_hbm.at[pl.ds((j * NW + wid) * CHUNK, CHUNK)], ibuf, sem_i)

    def do_gather(j, ibuf, sem_i, first):
        base = pl.multiple_of((j * NW + wid) * CHUNK, CHUNK)
        pltpu.make_async_copy(src_hbm.at[pl.ds(0, CHUNK)], ibuf, sem_i).wait()
        pltpu.async_copy(table.at[ibuf], rows, sem_g).wait()

        @pl.when(jnp.logical_not(first))
        def _():
            pltpu.make_async_copy(ov, o_hbm.at[pl.ds(0, CHUNK), :],
                                  sem_w).wait()

        def compact(i, _):
            for k in range(4):
                ov[i, pl.ds(k * 16, 16)] = rows[i, pl.ds(k * 16, 16)]
            return 0

        lax.fori_loop(0, CHUNK, compact, 0)
        pltpu.async_copy(ov, o_hbm.at[pl.ds(base, CHUNK), :], sem_w)

    start_idx(0, ia, sem_ia)

    def pbody(jj, _):
        j0 = jj * 2
        j1 = j0 + 1

        @pl.when(j1 < nj)
        def _():
            start_idx(j1, ib, sem_ib)

        do_gather(j0, ia, sem_ia, j0 == 0)

        @pl.when(j1 < nj)
        def _():
            @pl.when(j1 + 1 < nj)
            def _():
                start_idx(j1 + 1, ia, sem_ia)

            do_gather(j1, ib, sem_ib, jnp.bool_(False))

        return 0

    lax.fori_loop(0, (nj + 1) // 2, pbody, 0)
    pltpu.make_async_copy(ov, o_hbm.at[pl.ds(0, CHUNK), :], sem_w).wait()

  return gat


def _gather64(src, xg):
    return _make_gather64()(src, xg)


@functools.lru_cache(maxsize=None)
def _make_scatter1():
  @functools.partial(
      pl.kernel,
      out_type=jax.ShapeDtypeStruct((NPAD,), jnp.float32),
      mesh=plsc.VectorSubcoreMesh(**_SC_MESH),
      scratch_types=[
          pltpu.VMEM_SHARED((NPAD,), jnp.float32),
          pltpu.VMEM(((NJ_S + 1) * CHUNK,), jnp.int32),
          pltpu.VMEM((NJ_S + 1, CHUNK), jnp.int32),
          pltpu.VMEM((CHUNK,), jnp.float32),
          pltpu.VMEM((NPAD // NS,), jnp.float32),
          pltpu.SemaphoreType.DMA,
      ],
  )
  def scat1(dst_hbm, data_hbm, out_hbm, table, idx_big, idx2, data_v, zbuf, sem):
    c = lax.axis_index("c")
    s = lax.axis_index("s")
    zero16 = jnp.zeros((16,), jnp.float32)
    zn = NPAD // NS

    @pl.when(c == 0)
    def _():
        def zfill(i, _):
            zbuf[pl.ds(i * 16, 16)] = zero16
            return 0

        lax.fori_loop(0, zn // 16, zfill, 0)
        pltpu.sync_copy(zbuf, table.at[pl.ds(s * zn, zn)])

    plsc.subcore_barrier()

    @pl.when(c == 0)
    def _():
        nj = jnp.where(s < NJ_REM_S, NJ_S + 1, NJ_S)
        base0 = pl.multiple_of(
            (s * NJ_S + jnp.minimum(s, NJ_REM_S)) * CHUNK, CHUNK)
        _stage_idx_span(dst_hbm, idx_big, base0, NJ_S, s < NJ_REM_S)
        _fill_idx2(idx_big, idx2, nj)

        def body(j, _):
            base = pl.multiple_of(base0 + j * CHUNK, CHUNK)
            pltpu.sync_copy(data_hbm.at[pl.ds(base, CHUNK)], data_v)
            pltpu.async_copy(data_v, table.at[idx2.at[j]], sem, add=True).wait()
            return 0

        lax.fori_loop(0, nj, body, 0)

    plsc.subcore_barrier()

    @pl.when(c == 0)
    def _():
        pltpu.sync_copy(table.at[pl.ds(s * zn, zn)], zbuf)
        pltpu.sync_copy(zbuf, out_hbm.at[pl.ds(s * zn, zn)])

  return scat1


def _scatter1(dst, vals):
    return _make_scatter1()(dst, vals)


def kernel(edge_index, edge_vec, edge_len, r_max, fc1_w1, fc1_w2,
           fc2_w1, fc2_w2):
    src = edge_index[0]
    dst = edge_index[1]
    shT, embT = _edge_features(edge_vec.T, edge_len.reshape(1, E),
                               r_max.reshape(1, 1))
    sh = shT.T
    emb = embT.T
    sh8 = sh.reshape(E // 8, 128)
    g1a, g1b = _scatter_gather16(dst, src, sh8)
    ef = _tp1_fused(emb, g1a, g1b, sh, fc1_w1, fc1_w2)
    x2a, x2b = _scatter96(dst, ef)
    xg = _gate(x2a, x2b)
    g2 = _gather64(src, xg)
    ef2 = _tp2_fused(emb, g2, fc2_w1, fc2_w2)
    out = _scatter1(dst, ef2.reshape(E))
    return out[:N].reshape(N, 1)


# speedup vs baseline: 2.7567x; 1.0959x over previous
import functools
import math

import jax
import jax.numpy as jnp
from jax import lax
from jax.experimental import pallas as pl
from jax.experimental.pallas import tpu as pltpu
from jax.experimental.pallas import tpu_sc as plsc

N = 10000
NPAD = 10240
E = 160000
INV_SQRT_NN = float(1.0 / math.sqrt(3.8))
INV_SQRT3 = float(1.0 / math.sqrt(3.0))

NC = 2
NS = 16
NW = NC * NS
CHUNK = 128
NCHUNK = E // CHUNK
NJ_W = NCHUNK // NW
NJ_REM = NCHUNK - NJ_W * NW
NJ_S = NCHUNK // NS
NJ_REM_S = NCHUNK - NJ_S * NS
ZROWS = 40

_SC_MESH = dict(core_axis_name="c", subcore_axis_name="s")


_BE = 16000


def _edge_body(rmax_ref, vec_ref, len_ref, sh_ref, emb_ref):
    v = vec_ref[...]
    x = v[0:1, :]
    y = v[1:2, :]
    z = v[2:3, :]
    r = jnp.sqrt(x * x + y * y + z * z)
    d = jnp.maximum(r, 1e-9)
    x = x / d
    y = y / d
    z = z / d
    s3 = math.sqrt(3.0)
    s15 = math.sqrt(15.0)
    sh = jnp.concatenate([
        jnp.ones_like(x),
        s3 * x, s3 * y, s3 * z,
        s15 * x * y,
        s15 * y * z,
        (math.sqrt(5.0) / 2.0) * (3.0 * z * z - 1.0),
        s15 * x * z,
        (s15 / 2.0) * (x * x - y * y),
        math.sqrt(35.0 / 8.0) * y * (3.0 * x * x - y * y),
        math.sqrt(105.0) * x * y * z,
        math.sqrt(21.0 / 8.0) * y * (4.0 * z * z - x * x - y * y),
        (math.sqrt(7.0) / 2.0) * z * (2.0 * z * z - 3.0 * x * x - 3.0 * y * y),
        math.sqrt(21.0 / 8.0) * x * (4.0 * z * z - x * x - y * y),
        (math.sqrt(105.0) / 2.0) * z * (x * x - y * y),
        math.sqrt(35.0 / 8.0) * x * (x * x - 3.0 * y * y),
    ], axis=0)
    sh_ref[...] = sh

    el = len_ref[...]
    rmax = rmax_ref[0, 0]
    step = rmax * 0.5
    cols = []
    for j in range(3):
        diff = (el - j * step) / step
        cols.append(jnp.exp(-(diff * diff)) * (1.0 / 1.12))
    emb_ref[...] = jnp.concatenate(cols, axis=0)


def _edge_features(edge_vecT, edge_lenT, rmax2):
    grid = E // _BE
    return pl.pallas_call(
        _edge_body,
        grid=(grid,),
        in_specs=[
            pl.BlockSpec((1, 1), lambda i: (0, 0)),
            pl.BlockSpec((3, _BE), lambda i: (0, i)),
            pl.BlockSpec((1, _BE), lambda i: (0, i)),
        ],
        out_specs=[
            pl.BlockSpec((16, _BE), lambda i: (0, i)),
            pl.BlockSpec((3, _BE), lambda i: (0, i)),
        ],
        out_shape=[
            jax.ShapeDtypeStruct((16, E), jnp.float32),
            jax.ShapeDtypeStruct((3, E), jnp.float32),
        ],
    )(rmax2, edge_vecT, edge_lenT)


_BT = 2000


def _tp1_body(emb_ref, g1a_ref, g1b_ref, sh_ref, w1_ref, w2_ref, ef_ref):
    emb = emb_ref[...]
    h = jnp.dot(emb, w1_ref[...], preferred_element_type=jnp.float32)
    h = jax.nn.relu(h * INV_SQRT3)
    w1e = jnp.dot(h, w2_ref[...],
                  preferred_element_type=jnp.float32) * (1.0 / 16.0)
    g = (g1a_ref[...] + g1b_ref[...]) * INV_SQRT_NN
    prod = g * sh_ref[...]
    d0 = prod[:, 0:1]
    d1 = jnp.sum(prod[:, 1:4], axis=1, keepdims=True)
    d2 = jnp.sum(prod[:, 4:9], axis=1, keepdims=True)
    d3 = jnp.sum(prod[:, 9:16], axis=1, keepdims=True)
    ef = (d0 * w1e[:, 0:96] + d1 * w1e[:, 96:192]
          + d2 * w1e[:, 192:288] + d3 * w1e[:, 288:384]) * 0.5
    ef_ref[...] = jnp.concatenate(
        [ef, jnp.zeros((ef.shape[0], 32), jnp.float32)], axis=1)


def _tp1_fused(emb, g1a, g1b, sh, fc1_w1, fc1_w2):
    grid = E // _BT
    return pl.pallas_call(
        _tp1_body,
        grid=(grid,),
        in_specs=[
            pl.BlockSpec((_BT, 3), lambda i: (i, 0)),
            pl.BlockSpec((_BT, 16), lambda i: (i, 0)),
            pl.BlockSpec((_BT, 16), lambda i: (i, 0)),
            pl.BlockSpec((_BT, 16), lambda i: (i, 0)),
            pl.BlockSpec((3, 256), lambda i: (0, 0)),
            pl.BlockSpec((256, 384), lambda i: (0, 0)),
        ],
        out_specs=pl.BlockSpec((_BT, 128), lambda i: (i, 0)),
        out_shape=jax.ShapeDtypeStruct((E, 128), jnp.float32),
    )(emb, g1a, g1b, sh, fc1_w1, fc1_w2)


_BN = 2000


def _gate_body(xa_ref, xb_ref, out_ref):
    x = (xa_ref[:, 0:96] + xb_ref[:, 0:96]) * INV_SQRT_NN
    scalars = jnp.concatenate(
        [jax.nn.relu(x[:, 0:16]), jnp.abs(x[:, 16:32])], axis=1)
    g = x[:, 32:64]
    gates = jnp.concatenate([
        jax.nn.relu(g[:, 0:8]), jnp.tanh(g[:, 8:16]),
        jax.nn.relu(g[:, 16:24]), jnp.tanh(g[:, 24:32])], axis=1)
    xg = jnp.concatenate([scalars, gates * x[:, 64:96]], axis=1)
    out_ref[...] = jnp.concatenate(
        [xg, jnp.zeros((xg.shape[0], 64), jnp.float32)], axis=1)


def _gate(x2a, x2b):
    grid = N // _BN
    return pl.pallas_call(
        _gate_body,
        grid=(grid,),
        in_specs=[
            pl.BlockSpec((_BN, 128), lambda i: (i, 0)),
            pl.BlockSpec((_BN, 128), lambda i: (i, 0)),
        ],
        out_specs=pl.BlockSpec((_BN, 128), lambda i: (i, 0)),
        out_shape=jax.ShapeDtypeStruct((N, 128), jnp.float32),
    )(x2a, x2b)


def _tp2_body(emb_ref, g2_ref, w1_ref, w2_ref, ef2_ref):
    emb = emb_ref[...]
    h = jnp.dot(emb, w1_ref[...], preferred_element_type=jnp.float32)
    h = jax.nn.relu(h * INV_SQRT3)
    w2e = jnp.dot(h, w2_ref[...],
                  preferred_element_type=jnp.float32) * (1.0 / 16.0)
    s = jnp.sum(g2_ref[...] * w2e, axis=1, keepdims=True)
    ef2_ref[...] = s * (0.125 * INV_SQRT_NN)


def _tp2_fused(emb, g2, fc2_w1, fc2_w2):
    grid = E // _BT
    return pl.pallas_call(
        _tp2_body,
        grid=(grid,),
        in_specs=[
            pl.BlockSpec((_BT, 3), lambda i: (i, 0)),
            pl.BlockSpec((_BT, 64), lambda i: (i, 0)),
            pl.BlockSpec((3, 256), lambda i: (0, 0)),
            pl.BlockSpec((256, 64), lambda i: (0, 0)),
        ],
        out_specs=pl.BlockSpec((_BT, 1), lambda i: (i, 0)),
        out_shape=jax.ShapeDtypeStruct((E, 1), jnp.float32),
    )(emb, g2, fc2_w1, fc2_w2)


def _zero_vmem_128(buf):
    zero16 = jnp.zeros((16,), jnp.float32)

    def zfill(r, _):
        for k in range(8):
            buf[r, pl.ds(k * 16, 16)] = zero16
        return 0

    lax.fori_loop(0, CHUNK, zfill, 0)


def _zero_table_from(table, zsrc, s):
    @pl.when(s < 10)
    def _():
        for k in range(7):
            pltpu.sync_copy(zsrc, table.at[pl.ds(s * 1000 + k * 128, 128), :])
        pltpu.sync_copy(zsrc.at[pl.ds(0, 104), :],
                        table.at[pl.ds(s * 1000 + 896, 104), :])


def _stage_idx_span(idx_hbm, idx_big, base0, n_full, extra):
    pltpu.sync_copy(idx_hbm.at[pl.ds(base0, n_full * CHUNK)],
                    idx_big.at[pl.ds(0, n_full * CHUNK)])

    @pl.when(extra)
    def _():
        pltpu.sync_copy(idx_hbm.at[pl.ds(base0 + n_full * CHUNK, CHUNK)],
                        idx_big.at[pl.ds(n_full * CHUNK, CHUNK)])


def _fill_idx2(idx_big, idx2, nj):
    def fill(j, _):
        for k in range(CHUNK // 16):
            idx2[j, pl.ds(k * 16, 16)] = idx_big[pl.ds(j * CHUNK + k * 16, 16)]
        return 0

    lax.fori_loop(0, nj, fill, 0)


@functools.lru_cache(maxsize=None)
def _make_scatter_gather16():
  @functools.partial(
      pl.kernel,
      out_type=(jax.ShapeDtypeStruct((E, 16), jnp.float32),
                jax.ShapeDtypeStruct((E, 16), jnp.float32)),
      mesh=plsc.VectorSubcoreMesh(**_SC_MESH),
      scratch_types=[
          pltpu.VMEM_SHARED((N, 128), jnp.float32),
          pltpu.VMEM((CHUNK,), jnp.int32),
          pltpu.VMEM((CHUNK,), jnp.int32),
          pltpu.VMEM((1, CHUNK), jnp.int32),
          pltpu.VMEM((16, 128), jnp.float32),
          pltpu.VMEM((CHUNK, 128), jnp.float32),
          pltpu.VMEM((CHUNK, 16), jnp.float32),
          pltpu.SemaphoreType.DMA,
          pltpu.SemaphoreType.DMA,
          pltpu.SemaphoreType.DMA,
          pltpu.SemaphoreType.DMA,
      ],
  )
  def sg(dst_hbm, src_hbm, sh8_hbm, o0_hbm, o1_hbm,
         table, ia, ib, idx2, dc, data_v, ov,
         sem_ia, sem_ib, sem_s, sem_w):
    c = lax.axis_index("c")
    s = lax.axis_index("s")
    wid = s * NC + c

    _zero_vmem_128(data_v)
    _zero_table_from(table, data_v, s)
    plsc.subcore_barrier()

    nj = jnp.where(wid < NJ_REM, NJ_W + 1, NJ_W)

    def start_idx(base, ibuf, sem_i):
        pltpu.async_copy(dst_hbm.at[pl.ds(base, CHUNK)], ibuf, sem_i)

    def do_scatter(j, ibuf, sem_i):
        base = pl.multiple_of((j * NW + wid) * CHUNK, CHUNK)
        pltpu.make_async_copy(dst_hbm.at[pl.ds(0, CHUNK)], ibuf, sem_i).wait()
        for k in range(CHUNK // 16):
            idx2[0, pl.ds(k * 16, 16)] = ibuf[pl.ds(k * 16, 16)]
        b8 = pl.multiple_of(base // 8, 16)
        pltpu.sync_copy(sh8_hbm.at[pl.ds(b8, CHUNK // 8), :], dc)
        for r in range(CHUNK // 8):
            for k in range(8):
                data_v[r * 8 + k, pl.ds(0, 16)] = dc[r, pl.ds(k * 16, 16)]
        pltpu.async_copy(data_v, table.at[idx2.at[0]], sem_s, add=True).wait()

    start_idx(wid * CHUNK, ia, sem_ia)

    def pbody(jj, _):
        j0 = jj * 2
        j1 = j0 + 1

        @pl.when(j1 < nj)
        def _():
            start_idx((j1 * NW + wid) * CHUNK, ib, sem_ib)

        do_scatter(j0, ia, sem_ia)

        @pl.when(j1 < nj)
        def _():
            @pl.when(j1 + 1 < nj)
            def _():
                start_idx(((j1 + 1) * NW + wid) * CHUNK, ia, sem_ia)

            do_scatter(j1, ib, sem_ib)

        return 0

    lax.fori_loop(0, (nj + 1) // 2, pbody, 0)
    plsc.subcore_barrier()

    njg = jnp.where(s < NJ_REM_S, NJ_S + 1, NJ_S)

    def gstart_idx(j, ibuf, sem_i):
        pltpu.async_copy(
            src_hbm.at[pl.ds((j * NS + s) * CHUNK, CHUNK)], ibuf, sem_i)

    def do_gather(j, ibuf, sem_i, first):
        base = pl.multiple_of((j * NS + s) * CHUNK, CHUNK)
        pltpu.make_async_copy(src_hbm.at[pl.ds(0, CHUNK)], ibuf, sem_i).wait()
        pltpu.async_copy(table.at[ibuf], data_v, sem_s).wait()

        @pl.when(jnp.logical_not(first))
        def _():
            pltpu.make_async_copy(ov, o0_hbm.at[pl.ds(0, CHUNK), :],
                                  sem_w).wait()

        for i in range(CHUNK):
            ov[i, :] = data_v[i, pl.ds(0, 16)]

        @pl.when(c == 0)
        def _():
            pltpu.async_copy(ov, o0_hbm.at[pl.ds(base, CHUNK), :], sem_w)

        @pl.when(c == 1)
        def _():
            pltpu.async_copy(ov, o1_hbm.at[pl.ds(base, CHUNK), :], sem_w)

    gstart_idx(0, ia, sem_ia)

    def gp(jj, _):
        j0 = jj * 2
        j1 = j0 + 1

        @pl.when(j1 < njg)
        def _():
            gstart_idx(j1, ib, sem_ib)

        do_gather(j0, ia, sem_ia, j0 == 0)

        @pl.when(j1 < njg)
        def _():
            @pl.when(j1 + 1 < njg)
            def _():
                gstart_idx(j1 + 1, ia, sem_ia)

            do_gather(j1, ib, sem_ib, jnp.bool_(False))

        return 0

    lax.fori_loop(0, (njg + 1) // 2, gp, 0)
    pltpu.make_async_copy(ov, o0_hbm.at[pl.ds(0, CHUNK), :], sem_w).wait()

  return sg


def _scatter_gather16(dst, src, sh8):
    return _make_scatter_gather16()(dst, src, sh8)


@functools.lru_cache(maxsize=None)
def _make_scatter96():
  @functools.partial(
      pl.kernel,
      out_type=(jax.ShapeDtypeStruct((N, 128), jnp.float32),
                jax.ShapeDtypeStruct((N, 128), jnp.float32)),
      mesh=plsc.VectorSubcoreMesh(**_SC_MESH),
      scratch_types=[
          pltpu.VMEM_SHARED((N, 128), jnp.float32),
          pltpu.VMEM((CHUNK,), jnp.int32),
          pltpu.VMEM((CHUNK,), jnp.int32),
          pltpu.VMEM((1, CHUNK), jnp.int32),
          pltpu.VMEM((CHUNK, 128), jnp.float32),
          pltpu.SemaphoreType.DMA,
          pltpu.SemaphoreType.DMA,
          pltpu.SemaphoreType.DMA,
      ],
  )
  def scat(dst_hbm, ef_hbm, o0_hbm, o1_hbm,
           table, ia, ib, idx2, data_v, sem_ia, sem_ib, sem_s):
    c = lax.axis_index("c")
    s = lax.axis_index("s")
    wid = s * NC + c

    _zero_vmem_128(data_v)
    _zero_table_from(table, data_v, s)
    plsc.subcore_barrier()

    nj = jnp.where(wid < NJ_REM, NJ_W + 1, NJ_W)

    def start_idx(j, ibuf, sem_i):
        pltpu.async_copy(
            dst_hbm.at[pl.ds((j * NW + wid) * CHUNK, CHUNK)], ibuf, sem_i)

    def do_scatter(j, ibuf, sem_i):
        base = pl.multiple_of((j * NW + wid) * CHUNK, CHUNK)
        pltpu.make_async_copy(dst_hbm.at[pl.ds(0, CHUNK)], ibuf, sem_i).wait()
        for k in range(CHUNK // 16):
            idx2[0, pl.ds(k * 16, 16)] = ibuf[pl.ds(k * 16, 16)]
        pltpu.sync_copy(ef_hbm.at[pl.ds(base, CHUNK), :], data_v)
        pltpu.async_copy(data_v, table.at[idx2.at[0]], sem_s, add=True).wait()

    start_idx(0, ia, sem_ia)

    def pbody(jj, _):
        j0 = jj * 2
        j1 = j0 + 1

        @pl.when(j1 < nj)
        def _():
            start_idx(j1, ib, sem_ib)

        do_scatter(j0, ia, sem_ia)

        @pl.when(j1 < nj)
        def _():
            @pl.when(j1 + 1 < nj)
            def _():
                start_idx(j1 + 1, ia, sem_ia)

            do_scatter(j1, ib, sem_ib)

        return 0

    lax.fori_loop(0, (nj + 1) // 2, pbody, 0)
    plsc.subcore_barrier()

    @pl.when(jnp.logical_and(c == 0, s < 10))
    def _():
        pltpu.sync_copy(table.at[pl.ds(s * 1000, 1000), :],
                        o0_hbm.at[pl.ds(s * 1000, 1000), :])

    @pl.when(jnp.logical_and(c == 1, s < 10))
    def _():
        pltpu.sync_copy(table.at[pl.ds(s * 1000, 1000), :],
                        o1_hbm.at[pl.ds(s * 1000, 1000), :])

  return scat


def _scatter96(dst, ef):
    return _make_scatter96()(dst, ef)


@functools.lru_cache(maxsize=None)
def _make_gather64():
  @functools.partial(
      pl.kernel,
      out_type=jax.ShapeDtypeStruct((E, 64), jnp.float32),
      mesh=plsc.VectorSubcoreMesh(**_SC_MESH),
      scratch_types=[
          pltpu.VMEM_SHARED((N, 128), jnp.float32),
          pltpu.VMEM((CHUNK,), jnp.int32),
          pltpu.VMEM((CHUNK,), jnp.int32),
          pltpu.VMEM((CHUNK, 128), jnp.float32),
          pltpu.VMEM((CHUNK, 64), jnp.float32),
          pltpu.SemaphoreType.DMA,
          pltpu.SemaphoreType.DMA,
          pltpu.SemaphoreType.DMA,
          pltpu.SemaphoreType.DMA,
      ],
  )
  def gat(src_hbm, xg_hbm, o_hbm, table, ia, ib, rows, ov,
          sem_ia, sem_ib, sem_g, sem_w):
    c = lax.axis_index("c")
    s = lax.axis_index("s")
    wid = s * NC + c

    @pl.when(s < 10)
    def _():
        pltpu.sync_copy(xg_hbm.at[pl.ds(s * 1000, 1000), :],
                        table.at[pl.ds(s * 1000, 1000), :])

    plsc.subcore_barrier()

    nj = jnp.where(wid < NJ_REM, NJ_W + 1, NJ_W)

    def start_idx(j, ibuf, sem_i):
        pltpu.async_copy(
            src_hbm.at[pl.ds((j * NW + wid) * CHUNK, CHUNK)], ibuf, sem_i)

    def do_gather(j, ibuf, sem_i, first):
        base = pl.multiple_of((j * NW + wid) * CHUNK, CHUNK)
        pltpu.make_async_copy(src_hbm.at[pl.ds(0, CHUNK)], ibuf, sem_i).wait()
        pltpu.async_copy(table.at[ibuf], rows, sem_g).wait()

        @pl.when(jnp.logical_not(first))
        def _():
            pltpu.make_async_copy(ov, o_hbm.at[pl.ds(0, CHUNK), :],
                                  sem_w).wait()

        def compact(i, _):
            for k in range(4):
                ov[i, pl.ds(k * 16, 16)] = rows[i, pl.ds(k * 16, 16)]
            return 0

        lax.fori_loop(0, CHUNK, compact, 0)
        pltpu.async_copy(ov, o_hbm.at[pl.ds(base, CHUNK), :], sem_w)

    start_idx(0, ia, sem_ia)

    def pbody(jj, _):
        j0 = jj * 2
        j1 = j0 + 1

        @pl.when(j1 < nj)
        def _():
            start_idx(j1, ib, sem_ib)

        do_gather(j0, ia, sem_ia, j0 == 0)

        @pl.when(j1 < nj)
        def _():
            @pl.when(j1 + 1 < nj)
            def _():
                start_idx(j1 + 1, ia, sem_ia)

            do_gather(j1, ib, sem_ib, jnp.bool_(False))

        return 0

    lax.fori_loop(0, (nj + 1) // 2, pbody, 0)
    pltpu.make_async_copy(ov, o_hbm.at[pl.ds(0, CHUNK), :], sem_w).wait()

  return gat


def _gather64(src, xg):
    return _make_gather64()(src, xg)


@functools.lru_cache(maxsize=None)
def _make_scatter1():
  @functools.partial(
      pl.kernel,
      out_type=jax.ShapeDtypeStruct((NPAD,), jnp.float32),
      mesh=plsc.VectorSubcoreMesh(**_SC_MESH),
      scratch_types=[
          pltpu.VMEM_SHARED((NPAD,), jnp.float32),
          pltpu.VMEM(((NJ_S + 1) * CHUNK,), jnp.int32),
          pltpu.VMEM((NJ_S + 1, CHUNK), jnp.int32),
          pltpu.VMEM((CHUNK,), jnp.float32),
          pltpu.VMEM((NPAD // NS,), jnp.float32),
          pltpu.SemaphoreType.DMA,
      ],
  )
  def scat1(dst_hbm, data_hbm, out_hbm, table, idx_big, idx2, data_v, zbuf, sem):
    c = lax.axis_index("c")
    s = lax.axis_index("s")
    zero16 = jnp.zeros((16,), jnp.float32)
    zn = NPAD // NS

    @pl.when(c == 0)
    def _():
        def zfill(i, _):
            zbuf[pl.ds(i * 16, 16)] = zero16
            return 0

        lax.fori_loop(0, zn // 16, zfill, 0)
        pltpu.sync_copy(zbuf, table.at[pl.ds(s * zn, zn)])

    plsc.subcore_barrier()

    @pl.when(c == 0)
    def _():
        nj = jnp.where(s < NJ_REM_S, NJ_S + 1, NJ_S)
        base0 = pl.multiple_of(
            (s * NJ_S + jnp.minimum(s, NJ_REM_S)) * CHUNK, CHUNK)
        _stage_idx_span(dst_hbm, idx_big, base0, NJ_S, s < NJ_REM_S)
        _fill_idx2(idx_big, idx2, nj)

        def body(j, _):
            base = pl.multiple_of(base0 + j * CHUNK, CHUNK)
            pltpu.sync_copy(data_hbm.at[pl.ds(base, CHUNK)], data_v)
            pltpu.async_copy(data_v, table.at[idx2.at[j]], sem, add=True).wait()
            return 0

        lax.fori_loop(0, nj, body, 0)

    plsc.subcore_barrier()

    @pl.when(c == 0)
    def _():
        pltpu.sync_copy(table.at[pl.ds(s * zn, zn)], zbuf)
        pltpu.sync_copy(zbuf, out_hbm.at[pl.ds(s * zn, zn)])

  return scat1


def _scatter1(dst, vals):
    return _make_scatter1()(dst, vals)


def kernel(edge_index, edge_vec, edge_len, r_max, fc1_w1, fc1_w2,
           fc2_w1, fc2_w2):
    src = edge_index[0]
    dst = edge_index[1]
    shT, embT = _edge_features(edge_vec.T, edge_len.reshape(1, E),
                               r_max.reshape(1, 1))
    sh = shT.T
    emb = embT.T
    sh8 = sh.reshape(E // 8, 128)
    g1a, g1b = _scatter_gather16(dst, src, sh8)
    ef = _tp1_fused(emb, g1a, g1b, sh, fc1_w1, fc1_w2)
    x2a, x2b = _scatter96(dst, ef)
    xg = _gate(x2a, x2b)
    g2 = _gather64(src, xg)
    ef2 = _tp2_fused(emb, g2, fc2_w1, fc2_w2)
    out = _scatter1(dst, ef2.reshape(E))
    return out[:N].reshape(N, 1)
